# baseline (speedup 1.0000x reference)
import numpy as np
import jax
import jax.numpy as jnp
from jax.experimental import pallas as pl

_C, _T, _H, _W = 3, 32, 256, 256
_ALPHA = 4
_NSLOW = _T // _ALPHA
_IDX = np.linspace(0.0, _T - 1, _NSLOW).astype(np.int32)
assert all(int(np.searchsorted(_IDX, t, side="right")) - 1 == (7 * t + 6) // 31
           for t in range(_T))


def _slot(t):
    return ((_NSLOW - 1) * (t + 1) - 1) // (_T - 1)


def _body(in_ref, slow_ref, fast_ref):
    t = pl.program_id(1)
    fast_ref[...] = in_ref[...]
    sel = (t == int(_IDX[0]))
    for i in _IDX[1:]:
        sel |= (t == int(i))

    @pl.when(sel)
    def _():
        slow_ref[...] = in_ref[...]


def kernel(frames):
    slow, fast = pl.pallas_call(
        _body,
        grid=(_C, _T),
        in_specs=[pl.BlockSpec((1, 1, _H, _W), lambda c, t: (c, t, 0, 0))],
        out_specs=[
            pl.BlockSpec((1, 1, _H, _W), lambda c, t: (c, _slot(t), 0, 0)),
            pl.BlockSpec((1, 1, _H, _W), lambda c, t: (c, t, 0, 0)),
        ],
        out_shape=[
            jax.ShapeDtypeStruct((_C, _NSLOW, _H, _W), jnp.float32),
            jax.ShapeDtypeStruct((_C, _T, _H, _W), jnp.float32),
        ],
    )(frames)
    return (slow, fast)
